# Initial kernel scaffold; baseline (speedup 1.0000x reference)
import functools

import jax
import jax.numpy as jnp
from jax import lax
from jax.experimental import pallas as pl
from jax.experimental.pallas import tpu as pltpu
from jax.experimental.pallas import tpu_sc as plsc

_NR = 50000
_NP = 25000
_E = 400000
_H = 64
_BR = 512
_NRP = 50176
_NPP = 25088
_BLK = 128
_NBLK = _E // _BLK
_ZR = 784


def _seg_sum_call(tab_lo, tab_hi, src, dst, n_dst_pad, with_count):
    mesh = plsc.VectorSubcoreMesh(core_axis_name="c", subcore_axis_name="s")
    r16 = n_dst_pad // 16
    nz = r16 // _ZR

    out_type = [jax.ShapeDtypeStruct((n_dst_pad, 32), jnp.float32),
                jax.ShapeDtypeStruct((n_dst_pad, 32), jnp.float32)]
    scratch = [
        pltpu.VMEM_SHARED((n_dst_pad, 32), jnp.float32),
        pltpu.VMEM((_ZR, 32), jnp.float32),
        pltpu.VMEM((_BLK,), jnp.int32),
        pltpu.VMEM((_BLK,), jnp.int32),
        pltpu.VMEM((_BLK, 32), jnp.float32),
        pltpu.SemaphoreType.DMA,
    ]
    if with_count:
        out_type.append(jax.ShapeDtypeStruct((_NPP, 32), jnp.float32))
        scratch.append(pltpu.VMEM_SHARED((_NPP, 32), jnp.float32))
        scratch.append(pltpu.VMEM((_BLK, 32), jnp.float32))

    def body(tab_lo_ref, tab_hi_ref, src_ref, dst_ref, out_lo_ref, out_hi_ref,
             *rest):
        if with_count:
            (cnt_ref, acc, zbuf, srci, dsti, rows, sem, cacc, ones) = rest
        else:
            (acc, zbuf, srci, dsti, rows, sem) = rest
        c = lax.axis_index("c")
        s = lax.axis_index("s")

        z16 = jnp.zeros((16,), jnp.float32)

        def zb(i, car):
            zbuf[i, pl.ds(0, 16)] = z16
            zbuf[i, pl.ds(16, 16)] = z16
            return car
        lax.fori_loop(0, _ZR, zb, 0)
        if with_count:
            o16 = jnp.ones((16,), jnp.float32)

            def ob(i, car):
                ones[i, pl.ds(0, 16)] = o16
                ones[i, pl.ds(16, 16)] = o16
                return car
            lax.fori_loop(0, _BLK, ob, 0)

        for k in range(nz):
            pltpu.sync_copy(zbuf, acc.at[pl.ds(s * r16 + k * _ZR, _ZR)])
        if with_count:
            for k in range(_NPP // 16 // _ZR):
                pltpu.sync_copy(zbuf, cacc.at[pl.ds(s * (_NPP // 16) + k * _ZR, _ZR)])

        plsc.subcore_barrier()

        nb = (_NBLK - s + 15) // 16

        def make_loop(tab_ref):
            def lb(j, car):
                base = (s + j * 16) * _BLK
                pltpu.sync_copy(src_ref.at[pl.ds(base, _BLK)], srci)
                pltpu.sync_copy(dst_ref.at[pl.ds(base, _BLK)], dsti)
                pltpu.async_copy(tab_ref.at[srci], rows, sem).wait()
                pltpu.sync_copy(rows, acc.at[dsti], add=True)
                if with_count:
                    pltpu.sync_copy(ones, cacc.at[dsti], add=True)
                return car
            return lb

        @pl.when(c == 0)
        def _():
            lax.fori_loop(0, nb, make_loop(tab_lo_ref), 0)

        @pl.when(c == 1)
        def _():
            lax.fori_loop(0, nb, make_loop(tab_hi_ref), 0)

        plsc.subcore_barrier()

        @pl.when(c == 0)
        def _():
            pltpu.sync_copy(acc.at[pl.ds(s * r16, r16)],
                            out_lo_ref.at[pl.ds(s * r16, r16)])

        @pl.when(c == 1)
        def _():
            pltpu.sync_copy(acc.at[pl.ds(s * r16, r16)],
                            out_hi_ref.at[pl.ds(s * r16, r16)])

        if with_count:
            wid = s * 2 + c
            pltpu.sync_copy(cacc.at[pl.ds(wid * _ZR, _ZR)],
                            cnt_ref.at[pl.ds(wid * _ZR, _ZR)])

    fn = pl.kernel(body, out_type=tuple(out_type), mesh=mesh,
                   scratch_types=tuple(scratch))
    return fn(tab_lo, tab_hi, src, dst)


def _relu(x):
    return jnp.maximum(x, 0.0)


def _full(shape):
    return pl.BlockSpec(shape, lambda i: (0, 0))


def _rows(w):
    return pl.BlockSpec((_BR, w), lambda i: (i, 0))


def _featuregen_p(freq, flit, W_freq, b_freq, W_flit, b_flit, W_fh, b_fh):
    def kfn(fr, fl, wfr, bfr, wfl, bfl, wfh, bfh, lo, hi):
        ff = _relu(fr[...] * wfr[...] + bfr[...])
        lf = _relu(jnp.dot(fl[...], wfl[...],
                           preferred_element_type=jnp.float32) + bfl[...])
        feat = _relu(jnp.dot(jnp.concatenate([ff, lf], axis=1), wfh[...],
                             preferred_element_type=jnp.float32) + bfh[...])
        lo[...] = feat[:, :32]
        hi[...] = feat[:, 32:]

    return pl.pallas_call(
        kfn,
        grid=(_NPP // _BR,),
        in_specs=[_rows(1), _rows(32), _full((1, _H)), _full((1, _H)),
                  _full((32, _H)), _full((1, _H)), _full((2 * _H, _H)),
                  _full((1, _H))],
        out_specs=[_rows(32), _rows(32)],
        out_shape=[jax.ShapeDtypeStruct((_NPP, 32), jnp.float32)] * 2,
    )(freq, flit, W_freq, b_freq, W_flit, b_flit, W_fh, b_fh)


def _featuregen_r(op, W_op, b_op, W_fn, b_fn):
    def kfn(o, wo, bo, wf, bf, lo, hi):
        f1 = _relu(jnp.dot(o[...], wo[...],
                           preferred_element_type=jnp.float32) + bo[...])
        feat = _relu(jnp.dot(f1, wf[...],
                             preferred_element_type=jnp.float32) + bf[...])
        lo[...] = feat[:, :32]
        hi[...] = feat[:, 32:]

    return pl.pallas_call(
        kfn,
        grid=(_NRP // _BR,),
        in_specs=[_rows(4), _full((4, _H)), _full((1, _H)), _full((_H, _H)),
                  _full((1, _H))],
        out_specs=[_rows(32), _rows(32)],
        out_shape=[jax.ShapeDtypeStruct((_NRP, 32), jnp.float32)] * 2,
    )(op, W_op, b_op, W_fn, b_fn)


def _conv_r(rlo, rhi, h1lo, h1hi, h2lo, h2hi, W_r, b_r):
    def kfn(a, b, c, d, e, f, wr, br, lo, hi):
        h = jnp.concatenate([c[...], d[...], e[...], f[...]], axis=1)
        u = _relu(jnp.dot(h, wr[...],
                          preferred_element_type=jnp.float32) + br[...])
        lo[...] = a[...] + u[:, :32]
        hi[...] = b[...] + u[:, 32:]

    return pl.pallas_call(
        kfn,
        grid=(_NRP // _BR,),
        in_specs=[_rows(32)] * 6 + [_full((2 * _H, _H)), _full((1, _H))],
        out_specs=[_rows(32), _rows(32)],
        out_shape=[jax.ShapeDtypeStruct((_NRP, 32), jnp.float32)] * 2,
    )(rlo, rhi, h1lo, h1hi, h2lo, h2hi, W_r, b_r)


def _conv_p(plo, phi, slo, shi, cnt, W_p, b_p):
    def kfn(a, b, sl, sh, cn, wp, bp, lo, hi):
        d = jnp.maximum(cn[...][:, 0:1], 1.0)
        h = jnp.concatenate([sl[...], sh[...]], axis=1) / d
        u = _relu(jnp.dot(h, wp[...],
                          preferred_element_type=jnp.float32) + bp[...])
        lo[...] = a[...] + u[:, :32]
        hi[...] = b[...] + u[:, 32:]

    return pl.pallas_call(
        kfn,
        grid=(_NPP // _BR,),
        in_specs=[_rows(32)] * 4 + [_rows(32), _full((_H, _H)), _full((1, _H))],
        out_specs=[_rows(32), _rows(32)],
        out_shape=[jax.ShapeDtypeStruct((_NPP, 32), jnp.float32)] * 2,
    )(plo, phi, slo, shi, cnt, W_p, b_p)


def _readout(rlo, rhi, W_h1, b_h1, W_h2, b_h2, W_h3, b_h3):
    nblk = _NRP // _BR

    def kfn(a, b, w1, b1, w2, b2, w3, b3, out, accs, accm):
        i = pl.program_id(0)

        @pl.when(i == 0)
        def _():
            accs[...] = jnp.zeros_like(accs)
            accm[...] = jnp.full_like(accm, -jnp.inf)

        x = jnp.concatenate([a[...], b[...]], axis=1)
        rid = lax.broadcasted_iota(jnp.int32, (_BR, 1), 0) + i * _BR
        mask = rid < _NR
        xs = jnp.where(mask, x, 0.0)
        xm = jnp.where(mask, x, -jnp.inf)
        accs[...] = accs[...] + jnp.sum(xs, axis=0, keepdims=True)
        accm[...] = jnp.maximum(accm[...], jnp.max(xm, axis=0, keepdims=True))

        @pl.when(i == nblk - 1)
        def _():
            emb = jnp.concatenate([accs[...], accm[...]], axis=1)
            h = _relu(jnp.dot(emb, w1[...],
                              preferred_element_type=jnp.float32) + b1[...])
            h = _relu(jnp.dot(h, w2[...],
                              preferred_element_type=jnp.float32) + b2[...])
            out[...] = jnp.dot(h, w3[...],
                               preferred_element_type=jnp.float32) + b3[...]

    return pl.pallas_call(
        kfn,
        grid=(nblk,),
        in_specs=[_rows(32), _rows(32), _full((2 * _H, _H)), _full((1, _H)),
                  _full((_H, _H)), _full((1, _H)), _full((_H, 11)),
                  _full((1, 11))],
        out_specs=pl.BlockSpec((1, 11), lambda i: (0, 0)),
        out_shape=jax.ShapeDtypeStruct((1, 11), jnp.float32),
        scratch_shapes=[pltpu.VMEM((1, _H), jnp.float32),
                        pltpu.VMEM((1, _H), jnp.float32)],
    )(rlo, rhi, W_h1, b_h1, W_h2, b_h2, W_h3, b_h3)


def kernel(freq, flit, op_type, pass_src, pass_dst, transfer_src, transfer_dst,
           connect_src, connect_dst, W_freq, b_freq, W_flit, b_flit, W_fh, b_fh,
           W_op, b_op, W_fn, b_fn, W_r1, b_r1, W_p1, b_p1, W_r2, b_r2, W_p2,
           b_p2, W_h1, b_h1, W_h2, b_h2, W_h3, b_h3):
    freq = jnp.pad(freq, ((0, _NPP - _NP), (0, 0)))
    flit = jnp.pad(flit, ((0, _NPP - _NP), (0, 0)))
    op_type = jnp.pad(op_type, ((0, _NRP - _NR), (0, 0)))
    idx = [x.astype(jnp.int32) for x in
           (pass_src, pass_dst, transfer_src, transfer_dst,
            connect_src, connect_dst)]
    pass_src, pass_dst, transfer_src, transfer_dst, connect_src, connect_dst = idx
    r2d = lambda v: v.reshape(1, -1)

    plo, phi = _featuregen_p(freq, flit, W_freq, r2d(b_freq), W_flit,
                             r2d(b_flit), W_fh, r2d(b_fh))
    rlo, rhi = _featuregen_r(op_type, W_op, r2d(b_op), W_fn, r2d(b_fn))

    cnt = None
    for (W_r, b_r, W_p, b_p) in ((W_r1, b_r1, W_p1, b_p1),
                                 (W_r2, b_r2, W_p2, b_p2)):
        h1lo, h1hi = _seg_sum_call(plo, phi, pass_src, pass_dst, _NRP, False)
        h2lo, h2hi = _seg_sum_call(rlo, rhi, connect_src, connect_dst, _NRP,
                                   False)
        if cnt is None:
            slo, shi, cnt = _seg_sum_call(rlo, rhi, transfer_src, transfer_dst,
                                          _NPP, True)
        else:
            slo, shi = _seg_sum_call(rlo, rhi, transfer_src, transfer_dst,
                                     _NPP, False)
        nrlo, nrhi = _conv_r(rlo, rhi, h1lo, h1hi, h2lo, h2hi, W_r, r2d(b_r))
        nplo, nphi = _conv_p(plo, phi, slo, shi, cnt, W_p, r2d(b_p))
        rlo, rhi, plo, phi = nrlo, nrhi, nplo, nphi

    return _readout(rlo, rhi, W_h1, r2d(b_h1), W_h2, r2d(b_h2), W_h3,
                    r2d(b_h3))


# speedup vs baseline: 2.9234x; 2.9234x over previous
import functools

import jax
import jax.numpy as jnp
from jax import lax
from jax.experimental import pallas as pl
from jax.experimental.pallas import tpu as pltpu
from jax.experimental.pallas import tpu_sc as plsc

_NR = 50000
_NP = 25000
_E = 400000
_H = 64
_BR = 512
_NRP = 50176
_NPP = 25088
_BLK = 128
_NBLK = _E // _BLK
_ZR = 392


def _seg_sum_call(tab_lo, tab_hi, src, dst, n_dst_pad, with_count):
    mesh = plsc.VectorSubcoreMesh(core_axis_name="c", subcore_axis_name="s")
    r16 = n_dst_pad // 16
    nz = r16 // _ZR

    out_type = [jax.ShapeDtypeStruct((n_dst_pad, 32), jnp.float32),
                jax.ShapeDtypeStruct((n_dst_pad, 32), jnp.float32)]
    scratch = [
        pltpu.VMEM_SHARED((n_dst_pad, 32), jnp.float32),
        pltpu.VMEM((_ZR, 32), jnp.float32),
        pltpu.VMEM((_BLK,), jnp.int32),
        pltpu.VMEM((_BLK,), jnp.int32),
        pltpu.VMEM((_BLK, 32), jnp.float32),
        pltpu.SemaphoreType.DMA,
    ]
    if with_count:
        out_type.append(jax.ShapeDtypeStruct((_NPP, 32), jnp.float32))
        scratch.append(pltpu.VMEM_SHARED((_NPP, 32), jnp.float32))
        scratch.append(pltpu.VMEM((_BLK, 32), jnp.float32))

    def body(tab_lo_ref, tab_hi_ref, src_ref, dst_ref, out_lo_ref, out_hi_ref,
             *rest):
        if with_count:
            (cnt_ref, acc, zbuf, srci, dsti, rows, sem, cacc, ones) = rest
        else:
            (acc, zbuf, srci, dsti, rows, sem) = rest
        c = lax.axis_index("c")
        s = lax.axis_index("s")

        z16 = jnp.zeros((16,), jnp.float32)

        def zb(i, car):
            zbuf[i, pl.ds(0, 16)] = z16
            zbuf[i, pl.ds(16, 16)] = z16
            return car
        lax.fori_loop(0, _ZR, zb, 0)
        if with_count:
            o16 = jnp.ones((16,), jnp.float32)

            def ob(i, car):
                ones[i, pl.ds(0, 16)] = o16
                ones[i, pl.ds(16, 16)] = o16
                return car
            lax.fori_loop(0, _BLK, ob, 0)

        for k in range(nz):
            pltpu.sync_copy(zbuf, acc.at[pl.ds(s * r16 + k * _ZR, _ZR)])
        if with_count:
            for k in range(_NPP // 16 // _ZR):
                pltpu.sync_copy(zbuf, cacc.at[pl.ds(s * (_NPP // 16) + k * _ZR, _ZR)])

        plsc.subcore_barrier()

        nb = (_NBLK - s + 15) // 16

        def make_loop(tab_ref):
            def lb(j, car):
                base = (s + j * 16) * _BLK
                pltpu.sync_copy(src_ref.at[pl.ds(base, _BLK)], srci)
                pltpu.sync_copy(dst_ref.at[pl.ds(base, _BLK)], dsti)
                pltpu.async_copy(tab_ref.at[srci], rows, sem).wait()
                pltpu.sync_copy(rows, acc.at[dsti], add=True)
                if with_count:
                    pltpu.sync_copy(ones, cacc.at[dsti], add=True)
                return car
            return lb

        @pl.when(c == 0)
        def _():
            lax.fori_loop(0, nb, make_loop(tab_lo_ref), 0)

        @pl.when(c == 1)
        def _():
            lax.fori_loop(0, nb, make_loop(tab_hi_ref), 0)

        plsc.subcore_barrier()

        @pl.when(c == 0)
        def _():
            pltpu.sync_copy(acc.at[pl.ds(s * r16, r16)],
                            out_lo_ref.at[pl.ds(s * r16, r16)])

        @pl.when(c == 1)
        def _():
            pltpu.sync_copy(acc.at[pl.ds(s * r16, r16)],
                            out_hi_ref.at[pl.ds(s * r16, r16)])

        if with_count:
            wid = s * 2 + c
            cw = _NPP // 32
            pltpu.sync_copy(cacc.at[pl.ds(wid * cw, cw)],
                            cnt_ref.at[pl.ds(wid * cw, cw)])

    fn = pl.kernel(body, out_type=tuple(out_type), mesh=mesh,
                   scratch_types=tuple(scratch),
                   compiler_params=pltpu.CompilerParams(
                       use_tc_tiling_on_sc=False))
    return fn(tab_lo, tab_hi, src, dst)


def _relu(x):
    return jnp.maximum(x, 0.0)


def _full(shape):
    return pl.BlockSpec(shape, lambda i: (0, 0))


def _rows(w):
    return pl.BlockSpec((_BR, w), lambda i: (i, 0))


def _featuregen_p(freq, flit, W_freq, b_freq, W_flit, b_flit, W_fh, b_fh):
    def kfn(fr, fl, wfr, bfr, wfl, bfl, wfh, bfh, lo, hi):
        ff = _relu(fr[...] * wfr[...] + bfr[...])
        lf = _relu(jnp.dot(fl[...], wfl[...],
                           preferred_element_type=jnp.float32) + bfl[...])
        feat = _relu(jnp.dot(jnp.concatenate([ff, lf], axis=1), wfh[...],
                             preferred_element_type=jnp.float32) + bfh[...])
        lo[...] = feat[:, :32]
        hi[...] = feat[:, 32:]

    return pl.pallas_call(
        kfn,
        grid=(_NPP // _BR,),
        in_specs=[_rows(1), _rows(32), _full((1, _H)), _full((1, _H)),
                  _full((32, _H)), _full((1, _H)), _full((2 * _H, _H)),
                  _full((1, _H))],
        out_specs=[_rows(32), _rows(32)],
        out_shape=[jax.ShapeDtypeStruct((_NPP, 32), jnp.float32)] * 2,
    )(freq, flit, W_freq, b_freq, W_flit, b_flit, W_fh, b_fh)


def _featuregen_r(op, W_op, b_op, W_fn, b_fn):
    def kfn(o, wo, bo, wf, bf, lo, hi):
        f1 = _relu(jnp.dot(o[...], wo[...],
                           preferred_element_type=jnp.float32) + bo[...])
        feat = _relu(jnp.dot(f1, wf[...],
                             preferred_element_type=jnp.float32) + bf[...])
        lo[...] = feat[:, :32]
        hi[...] = feat[:, 32:]

    return pl.pallas_call(
        kfn,
        grid=(_NRP // _BR,),
        in_specs=[_rows(4), _full((4, _H)), _full((1, _H)), _full((_H, _H)),
                  _full((1, _H))],
        out_specs=[_rows(32), _rows(32)],
        out_shape=[jax.ShapeDtypeStruct((_NRP, 32), jnp.float32)] * 2,
    )(op, W_op, b_op, W_fn, b_fn)


def _conv_r(rlo, rhi, h1lo, h1hi, h2lo, h2hi, W_r, b_r):
    def kfn(a, b, c, d, e, f, wr, br, lo, hi):
        h = jnp.concatenate([c[...], d[...], e[...], f[...]], axis=1)
        u = _relu(jnp.dot(h, wr[...],
                          preferred_element_type=jnp.float32) + br[...])
        lo[...] = a[...] + u[:, :32]
        hi[...] = b[...] + u[:, 32:]

    return pl.pallas_call(
        kfn,
        grid=(_NRP // _BR,),
        in_specs=[_rows(32)] * 6 + [_full((2 * _H, _H)), _full((1, _H))],
        out_specs=[_rows(32), _rows(32)],
        out_shape=[jax.ShapeDtypeStruct((_NRP, 32), jnp.float32)] * 2,
    )(rlo, rhi, h1lo, h1hi, h2lo, h2hi, W_r, b_r)


def _conv_p(plo, phi, slo, shi, cnt, W_p, b_p):
    def kfn(a, b, sl, sh, cn, wp, bp, lo, hi):
        d = jnp.maximum(cn[...][:, 0:1], 1.0)
        h = jnp.concatenate([sl[...], sh[...]], axis=1) / d
        u = _relu(jnp.dot(h, wp[...],
                          preferred_element_type=jnp.float32) + bp[...])
        lo[...] = a[...] + u[:, :32]
        hi[...] = b[...] + u[:, 32:]

    return pl.pallas_call(
        kfn,
        grid=(_NPP // _BR,),
        in_specs=[_rows(32)] * 4 + [_rows(32), _full((_H, _H)), _full((1, _H))],
        out_specs=[_rows(32), _rows(32)],
        out_shape=[jax.ShapeDtypeStruct((_NPP, 32), jnp.float32)] * 2,
    )(plo, phi, slo, shi, cnt, W_p, b_p)


def _readout(rlo, rhi, W_h1, b_h1, W_h2, b_h2, W_h3, b_h3):
    nblk = _NRP // _BR

    def kfn(a, b, w1, b1, w2, b2, w3, b3, out, accs, accm):
        i = pl.program_id(0)

        @pl.when(i == 0)
        def _():
            accs[...] = jnp.zeros_like(accs)
            accm[...] = jnp.full_like(accm, -jnp.inf)

        x = jnp.concatenate([a[...], b[...]], axis=1)
        rid = lax.broadcasted_iota(jnp.int32, (_BR, 1), 0) + i * _BR
        mask = rid < _NR
        xs = jnp.where(mask, x, 0.0)
        xm = jnp.where(mask, x, -jnp.inf)
        accs[...] = accs[...] + jnp.sum(xs, axis=0, keepdims=True)
        accm[...] = jnp.maximum(accm[...], jnp.max(xm, axis=0, keepdims=True))

        @pl.when(i == nblk - 1)
        def _():
            emb = jnp.concatenate([accs[...], accm[...]], axis=1)
            h = _relu(jnp.dot(emb, w1[...],
                              preferred_element_type=jnp.float32) + b1[...])
            h = _relu(jnp.dot(h, w2[...],
                              preferred_element_type=jnp.float32) + b2[...])
            out[...] = jnp.dot(h, w3[...],
                               preferred_element_type=jnp.float32) + b3[...]

    return pl.pallas_call(
        kfn,
        grid=(nblk,),
        in_specs=[_rows(32), _rows(32), _full((2 * _H, _H)), _full((1, _H)),
                  _full((_H, _H)), _full((1, _H)), _full((_H, 11)),
                  _full((1, 11))],
        out_specs=pl.BlockSpec((1, 11), lambda i: (0, 0)),
        out_shape=jax.ShapeDtypeStruct((1, 11), jnp.float32),
        scratch_shapes=[pltpu.VMEM((1, _H), jnp.float32),
                        pltpu.VMEM((1, _H), jnp.float32)],
    )(rlo, rhi, W_h1, b_h1, W_h2, b_h2, W_h3, b_h3)


def kernel(freq, flit, op_type, pass_src, pass_dst, transfer_src, transfer_dst,
           connect_src, connect_dst, W_freq, b_freq, W_flit, b_flit, W_fh, b_fh,
           W_op, b_op, W_fn, b_fn, W_r1, b_r1, W_p1, b_p1, W_r2, b_r2, W_p2,
           b_p2, W_h1, b_h1, W_h2, b_h2, W_h3, b_h3):
    freq = jnp.pad(freq, ((0, _NPP - _NP), (0, 0)))
    flit = jnp.pad(flit, ((0, _NPP - _NP), (0, 0)))
    op_type = jnp.pad(op_type, ((0, _NRP - _NR), (0, 0)))
    idx = [x.astype(jnp.int32) for x in
           (pass_src, pass_dst, transfer_src, transfer_dst,
            connect_src, connect_dst)]
    pass_src, pass_dst, transfer_src, transfer_dst, connect_src, connect_dst = idx
    r2d = lambda v: v.reshape(1, -1)

    plo, phi = _featuregen_p(freq, flit, W_freq, r2d(b_freq), W_flit,
                             r2d(b_flit), W_fh, r2d(b_fh))
    rlo, rhi = _featuregen_r(op_type, W_op, r2d(b_op), W_fn, r2d(b_fn))

    cnt = None
    for (W_r, b_r, W_p, b_p) in ((W_r1, b_r1, W_p1, b_p1),
                                 (W_r2, b_r2, W_p2, b_p2)):
        h1lo, h1hi = _seg_sum_call(plo, phi, pass_src, pass_dst, _NRP, False)
        h2lo, h2hi = _seg_sum_call(rlo, rhi, connect_src, connect_dst, _NRP,
                                   False)
        if cnt is None:
            slo, shi, cnt = _seg_sum_call(rlo, rhi, transfer_src, transfer_dst,
                                          _NPP, True)
        else:
            slo, shi = _seg_sum_call(rlo, rhi, transfer_src, transfer_dst,
                                     _NPP, False)
        nrlo, nrhi = _conv_r(rlo, rhi, h1lo, h1hi, h2lo, h2hi, W_r, r2d(b_r))
        nplo, nphi = _conv_p(plo, phi, slo, shi, cnt, W_p, r2d(b_p))
        rlo, rhi, plo, phi = nrlo, nrhi, nplo, nphi

    return _readout(rlo, rhi, W_h1, r2d(b_h1), W_h2, r2d(b_h2), W_h3,
                    r2d(b_h3))


# speedup vs baseline: 4.4674x; 1.5281x over previous
import functools

import jax
import jax.numpy as jnp
from jax import lax
from jax.experimental import pallas as pl
from jax.experimental.pallas import tpu as pltpu
from jax.experimental.pallas import tpu_sc as plsc

_NR = 50000
_NP = 25000
_E = 400000
_H = 64
_BR = 512
_NRP = 50176
_NPP = 25088
_BLK = 128
_EP = 409600
_NBLK = _EP // _BLK
_TBLK = _NBLK // 16
_NCH2 = _TBLK // 8
_ZR = 196


def _seg_sum_call(tab_lo, tab_hi, src, dst, n_dst_pad, with_count):
    mesh = plsc.VectorSubcoreMesh(core_axis_name="c", subcore_axis_name="s")
    r16 = n_dst_pad // 16
    nz = r16 // _ZR

    out_type = [jax.ShapeDtypeStruct((n_dst_pad, 32), jnp.float32),
                jax.ShapeDtypeStruct((n_dst_pad, 32), jnp.float32)]
    scratch = [
        pltpu.VMEM_SHARED((n_dst_pad, 32), jnp.float32),
        pltpu.VMEM((_ZR, 32), jnp.float32),
        pltpu.VMEM((2, 4, _BLK), jnp.int32),
        pltpu.VMEM((2, 4, _BLK), jnp.int32),
        pltpu.VMEM((4, _BLK, 32), jnp.float32),
        pltpu.SemaphoreType.DMA,
        pltpu.SemaphoreType.DMA,
        pltpu.SemaphoreType.DMA,
        pltpu.SemaphoreType.DMA,
    ]
    if with_count:
        out_type.append(jax.ShapeDtypeStruct((_NPP, 32), jnp.float32))
        scratch.append(pltpu.VMEM_SHARED((_NPP, 32), jnp.float32))
        scratch.append(pltpu.VMEM((_BLK, 32), jnp.float32))

    def body(tab_lo_ref, tab_hi_ref, src_ref, dst_ref, out_lo_ref, out_hi_ref,
             *rest):
        if with_count:
            (cnt_ref, acc, zbuf, srci, dsti, rows,
             g0, g1, g2, g3, cacc, ones) = rest
        else:
            (acc, zbuf, srci, dsti, rows, g0, g1, g2, g3) = rest
        gsem = (g0, g1, g2, g3)
        c = lax.axis_index("c")
        s = lax.axis_index("s")

        z16 = jnp.zeros((16,), jnp.float32)

        def zb(i, car):
            zbuf[i, pl.ds(0, 16)] = z16
            zbuf[i, pl.ds(16, 16)] = z16
            return car
        lax.fori_loop(0, _ZR, zb, 0)
        if with_count:
            o16 = jnp.ones((16,), jnp.float32)

            def ob(i, car):
                ones[i, pl.ds(0, 16)] = o16
                ones[i, pl.ds(16, 16)] = o16
                return car
            lax.fori_loop(0, _BLK, ob, 0)

        for k in range(nz):
            pltpu.sync_copy(zbuf, acc.at[pl.ds(s * r16 + k * _ZR, _ZR)])
        if with_count:
            for k in range(_NPP // 16 // _ZR):
                pltpu.sync_copy(zbuf, cacc.at[pl.ds(s * (_NPP // 16) + k * _ZR, _ZR)])

        plsc.subcore_barrier()

        tb = s * _TBLK

        def run(tab_ref):
            def load_idx(ch, buf):
                pltpu.sync_copy(src_ref.at[pl.ds(tb + ch * 4, 4)], srci.at[buf])
                pltpu.sync_copy(dst_ref.at[pl.ds(tb + ch * 4, 4)], dsti.at[buf])

            def fire(buf, b):
                pltpu.async_copy(tab_ref.at[srci.at[buf, b]], rows.at[b],
                                 gsem[b])

            def drain(buf, b):
                pltpu.make_async_copy(tab_ref.at[srci.at[buf, b]], rows.at[b],
                                      gsem[b]).wait()
                pltpu.sync_copy(rows.at[b], acc.at[dsti.at[buf, b]], add=True)
                if with_count:
                    pltpu.sync_copy(ones, cacc.at[dsti.at[buf, b]], add=True)

            load_idx(0, 0)
            for b in range(4):
                fire(0, b)

            def chunk2(c2, car):
                load_idx(2 * c2 + 1, 1)
                for b in range(4):
                    drain(0, b)
                    fire(1, b)
                @pl.when(c2 < _NCH2 - 1)
                def _():
                    load_idx(2 * c2 + 2, 0)
                for b in range(4):
                    drain(1, b)

                    @pl.when(c2 < _NCH2 - 1)
                    def _(b=b):
                        fire(0, b)
                return car
            lax.fori_loop(0, _NCH2, chunk2, 0)

        @pl.when(c == 0)
        def _():
            run(tab_lo_ref)

        @pl.when(c == 1)
        def _():
            run(tab_hi_ref)

        plsc.subcore_barrier()

        @pl.when(c == 0)
        def _():
            pltpu.sync_copy(acc.at[pl.ds(s * r16, r16)],
                            out_lo_ref.at[pl.ds(s * r16, r16)])

        @pl.when(c == 1)
        def _():
            pltpu.sync_copy(acc.at[pl.ds(s * r16, r16)],
                            out_hi_ref.at[pl.ds(s * r16, r16)])

        if with_count:
            wid = s * 2 + c
            cw = _NPP // 32
            pltpu.sync_copy(cacc.at[pl.ds(wid * cw, cw)],
                            cnt_ref.at[pl.ds(wid * cw, cw)])

    fn = pl.kernel(body, out_type=tuple(out_type), mesh=mesh,
                   scratch_types=tuple(scratch),
                   compiler_params=pltpu.CompilerParams(
                       use_tc_tiling_on_sc=False))
    return fn(tab_lo, tab_hi, src, dst)


def _relu(x):
    return jnp.maximum(x, 0.0)


def _full(shape):
    return pl.BlockSpec(shape, lambda i: (0, 0))


def _rows(w):
    return pl.BlockSpec((_BR, w), lambda i: (i, 0))


def _featuregen_p(freq, flit, W_freq, b_freq, W_flit, b_flit, W_fh, b_fh):
    def kfn(fr, fl, wfr, bfr, wfl, bfl, wfh, bfh, lo, hi):
        ff = _relu(fr[...] * wfr[...] + bfr[...])
        lf = _relu(jnp.dot(fl[...], wfl[...],
                           preferred_element_type=jnp.float32) + bfl[...])
        feat = _relu(jnp.dot(jnp.concatenate([ff, lf], axis=1), wfh[...],
                             preferred_element_type=jnp.float32) + bfh[...])
        lo[...] = feat[:, :32]
        hi[...] = feat[:, 32:]

    return pl.pallas_call(
        kfn,
        grid=(_NPP // _BR,),
        in_specs=[_rows(1), _rows(32), _full((1, _H)), _full((1, _H)),
                  _full((32, _H)), _full((1, _H)), _full((2 * _H, _H)),
                  _full((1, _H))],
        out_specs=[_rows(32), _rows(32)],
        out_shape=[jax.ShapeDtypeStruct((_NPP, 32), jnp.float32)] * 2,
    )(freq, flit, W_freq, b_freq, W_flit, b_flit, W_fh, b_fh)


def _featuregen_r(op, W_op, b_op, W_fn, b_fn):
    def kfn(o, wo, bo, wf, bf, lo, hi):
        f1 = _relu(jnp.dot(o[...], wo[...],
                           preferred_element_type=jnp.float32) + bo[...])
        feat = _relu(jnp.dot(f1, wf[...],
                             preferred_element_type=jnp.float32) + bf[...])
        lo[...] = feat[:, :32]
        hi[...] = feat[:, 32:]

    return pl.pallas_call(
        kfn,
        grid=(_NRP // _BR,),
        in_specs=[_rows(4), _full((4, _H)), _full((1, _H)), _full((_H, _H)),
                  _full((1, _H))],
        out_specs=[_rows(32), _rows(32)],
        out_shape=[jax.ShapeDtypeStruct((_NRP, 32), jnp.float32)] * 2,
    )(op, W_op, b_op, W_fn, b_fn)


def _conv_r(rlo, rhi, h1lo, h1hi, h2lo, h2hi, W_r, b_r):
    def kfn(a, b, c, d, e, f, wr, br, lo, hi):
        h = jnp.concatenate([c[...], d[...], e[...], f[...]], axis=1)
        u = _relu(jnp.dot(h, wr[...],
                          preferred_element_type=jnp.float32) + br[...])
        lo[...] = a[...] + u[:, :32]
        hi[...] = b[...] + u[:, 32:]

    return pl.pallas_call(
        kfn,
        grid=(_NRP // _BR,),
        in_specs=[_rows(32)] * 6 + [_full((2 * _H, _H)), _full((1, _H))],
        out_specs=[_rows(32), _rows(32)],
        out_shape=[jax.ShapeDtypeStruct((_NRP, 32), jnp.float32)] * 2,
    )(rlo, rhi, h1lo, h1hi, h2lo, h2hi, W_r, b_r)


def _conv_p(plo, phi, slo, shi, cnt, W_p, b_p):
    def kfn(a, b, sl, sh, cn, wp, bp, lo, hi):
        d = jnp.maximum(cn[...][:, 0:1], 1.0)
        h = jnp.concatenate([sl[...], sh[...]], axis=1) / d
        u = _relu(jnp.dot(h, wp[...],
                          preferred_element_type=jnp.float32) + bp[...])
        lo[...] = a[...] + u[:, :32]
        hi[...] = b[...] + u[:, 32:]

    return pl.pallas_call(
        kfn,
        grid=(_NPP // _BR,),
        in_specs=[_rows(32)] * 4 + [_rows(32), _full((_H, _H)), _full((1, _H))],
        out_specs=[_rows(32), _rows(32)],
        out_shape=[jax.ShapeDtypeStruct((_NPP, 32), jnp.float32)] * 2,
    )(plo, phi, slo, shi, cnt, W_p, b_p)


def _readout(rlo, rhi, W_h1, b_h1, W_h2, b_h2, W_h3, b_h3):
    nblk = _NRP // _BR

    def kfn(a, b, w1, b1, w2, b2, w3, b3, out, accs, accm):
        i = pl.program_id(0)

        @pl.when(i == 0)
        def _():
            accs[...] = jnp.zeros_like(accs)
            accm[...] = jnp.full_like(accm, -jnp.inf)

        x = jnp.concatenate([a[...], b[...]], axis=1)
        rid = lax.broadcasted_iota(jnp.int32, (_BR, 1), 0) + i * _BR
        mask = rid < _NR
        xs = jnp.where(mask, x, 0.0)
        xm = jnp.where(mask, x, -jnp.inf)
        accs[...] = accs[...] + jnp.sum(xs, axis=0, keepdims=True)
        accm[...] = jnp.maximum(accm[...], jnp.max(xm, axis=0, keepdims=True))

        @pl.when(i == nblk - 1)
        def _():
            emb = jnp.concatenate([accs[...], accm[...]], axis=1)
            h = _relu(jnp.dot(emb, w1[...],
                              preferred_element_type=jnp.float32) + b1[...])
            h = _relu(jnp.dot(h, w2[...],
                              preferred_element_type=jnp.float32) + b2[...])
            out[...] = jnp.dot(h, w3[...],
                               preferred_element_type=jnp.float32) + b3[...]

    return pl.pallas_call(
        kfn,
        grid=(nblk,),
        in_specs=[_rows(32), _rows(32), _full((2 * _H, _H)), _full((1, _H)),
                  _full((_H, _H)), _full((1, _H)), _full((_H, 11)),
                  _full((1, 11))],
        out_specs=pl.BlockSpec((1, 11), lambda i: (0, 0)),
        out_shape=jax.ShapeDtypeStruct((1, 11), jnp.float32),
        scratch_shapes=[pltpu.VMEM((1, _H), jnp.float32),
                        pltpu.VMEM((1, _H), jnp.float32)],
    )(rlo, rhi, W_h1, b_h1, W_h2, b_h2, W_h3, b_h3)


def kernel(freq, flit, op_type, pass_src, pass_dst, transfer_src, transfer_dst,
           connect_src, connect_dst, W_freq, b_freq, W_flit, b_flit, W_fh, b_fh,
           W_op, b_op, W_fn, b_fn, W_r1, b_r1, W_p1, b_p1, W_r2, b_r2, W_p2,
           b_p2, W_h1, b_h1, W_h2, b_h2, W_h3, b_h3):
    freq = jnp.pad(freq, ((0, _NPP - _NP), (0, 0)))
    flit = jnp.pad(flit, ((0, _NPP - _NP), (0, 0)))
    op_type = jnp.pad(op_type, ((0, _NRP - _NR), (0, 0)))
    def pad_idx(a, fill):
        a = a.astype(jnp.int32)
        return jnp.concatenate(
            [a, jnp.full((_EP - _E,), fill, jnp.int32)]).reshape(_NBLK, _BLK)

    pass_src = pad_idx(pass_src, 0)
    pass_dst = pad_idx(pass_dst, _NR)
    transfer_src = pad_idx(transfer_src, 0)
    transfer_dst = pad_idx(transfer_dst, _NP)
    connect_src = pad_idx(connect_src, 0)
    connect_dst = pad_idx(connect_dst, _NR)
    r2d = lambda v: v.reshape(1, -1)

    plo, phi = _featuregen_p(freq, flit, W_freq, r2d(b_freq), W_flit,
                             r2d(b_flit), W_fh, r2d(b_fh))
    rlo, rhi = _featuregen_r(op_type, W_op, r2d(b_op), W_fn, r2d(b_fn))

    cnt = None
    for (W_r, b_r, W_p, b_p) in ((W_r1, b_r1, W_p1, b_p1),
                                 (W_r2, b_r2, W_p2, b_p2)):
        h1lo, h1hi = _seg_sum_call(plo, phi, pass_src, pass_dst, _NRP, False)
        h2lo, h2hi = _seg_sum_call(rlo, rhi, connect_src, connect_dst, _NRP,
                                   False)
        if cnt is None:
            slo, shi, cnt = _seg_sum_call(rlo, rhi, transfer_src, transfer_dst,
                                          _NPP, True)
        else:
            slo, shi = _seg_sum_call(rlo, rhi, transfer_src, transfer_dst,
                                     _NPP, False)
        nrlo, nrhi = _conv_r(rlo, rhi, h1lo, h1hi, h2lo, h2hi, W_r, r2d(b_r))
        nplo, nphi = _conv_p(plo, phi, slo, shi, cnt, W_p, r2d(b_p))
        rlo, rhi, plo, phi = nrlo, nrhi, nplo, nphi

    return _readout(rlo, rhi, W_h1, r2d(b_h1), W_h2, r2d(b_h2), W_h3,
                    r2d(b_h3))
